# Initial kernel scaffold; baseline (speedup 1.0000x reference)
import functools

import jax
import jax.numpy as jnp
from jax import lax
from jax.experimental import pallas as pl
from jax.experimental.pallas import tpu as pltpu
from jax.experimental.pallas import tpu_sc as plsc

N = 10000
E = 320000
D = 128
EPS = 1e-07

NC = 2
NS = 16
NW = NC * NS
EPW = E // NW
CHUNK = 80
NCHUNK = EPW // CHUNK
RPS = N // NS
ZROWS = 125


def _zero_buf(ref, rows, width):
  def body(i, _):
    for j in range(width // 16):
      ref[i, pl.ds(j * 16, 16)] = jnp.zeros((16,), jnp.float32)
    return 0
  lax.fori_loop(0, rows, body, 0)


def _fill_ones(ref, rows, width):
  def body(i, _):
    for j in range(width // 16):
      ref[i, pl.ds(j * 16, 16)] = jnp.ones((16,), jnp.float32)
    return 0
  lax.fori_loop(0, rows, body, 0)


def _make_segsum(width, with_cnt):
  mesh = plsc.VectorSubcoreMesh(core_axis_name="c", subcore_axis_name="s")
  out_type = [jax.ShapeDtypeStruct((NC, N, width), jnp.float32)]
  if with_cnt:
    out_type.append(jax.ShapeDtypeStruct((NC, N, 16), jnp.float32))
  scratch = [
      pltpu.VMEM_SHARED((N, width), jnp.float32),
      pltpu.VMEM((CHUNK,), jnp.int32),
      pltpu.VMEM((CHUNK,), jnp.int32),
      pltpu.VMEM((CHUNK, width), jnp.float32),
      pltpu.VMEM((ZROWS, width), jnp.float32),
      pltpu.SemaphoreType.DMA,
  ]
  if with_cnt:
    scratch += [
        pltpu.VMEM_SHARED((N, 16), jnp.float32),
        pltpu.VMEM((CHUNK, 16), jnp.float32),
        pltpu.VMEM((ZROWS, 16), jnp.float32),
    ]

  def body(table_hbm, gidx_hbm, sidx_hbm, out_hbm, *rest):
    if with_cnt:
      (cnt_hbm, acc_sh, gi_v, si_v, rows_v, zbuf, sem,
       cnt_sh, ones_v, zbuf16) = rest
    else:
      (acc_sh, gi_v, si_v, rows_v, zbuf, sem) = rest
    c = lax.axis_index("c")
    s = lax.axis_index("s")
    wid = c * NS + s

    _zero_buf(zbuf, ZROWS, width)
    row0 = s * RPS
    for r in range(RPS // ZROWS):
      pltpu.sync_copy(zbuf, acc_sh.at[pl.ds(row0 + r * ZROWS, ZROWS)])
    if with_cnt:
      _zero_buf(zbuf16, ZROWS, 16)
      _fill_ones(ones_v, CHUNK, 16)
      for r in range(RPS // ZROWS):
        pltpu.sync_copy(zbuf16, cnt_sh.at[pl.ds(row0 + r * ZROWS, ZROWS)])
    plsc.subcore_barrier()

    base = wid * EPW

    def chunk_body(i, _):
      off = pl.multiple_of(base + i * CHUNK, 8)
      pltpu.sync_copy(gidx_hbm.at[pl.ds(off, CHUNK)], gi_v)
      pltpu.sync_copy(sidx_hbm.at[pl.ds(off, CHUNK)], si_v)
      pltpu.async_copy(table_hbm.at[gi_v], rows_v, sem).wait()
      pltpu.sync_copy(rows_v, acc_sh.at[si_v], add=True)
      if with_cnt:
        pltpu.sync_copy(ones_v, cnt_sh.at[si_v], add=True)
      return 0

    lax.fori_loop(0, NCHUNK, chunk_body, 0)
    plsc.subcore_barrier()

    for r in range(RPS // ZROWS):
      rr = row0 + r * ZROWS
      pltpu.sync_copy(acc_sh.at[pl.ds(rr, ZROWS)], zbuf)
      pltpu.sync_copy(zbuf, out_hbm.at[c, pl.ds(rr, ZROWS)])
      if with_cnt:
        pltpu.sync_copy(cnt_sh.at[pl.ds(rr, ZROWS)], zbuf16)
        pltpu.sync_copy(zbuf16, cnt_hbm.at[c, pl.ds(rr, ZROWS)])

  return pl.kernel(body, out_type=tuple(out_type), mesh=mesh,
                   scratch_types=scratch)


_segsum128_cnt = _make_segsum(D, True)
_segsum128 = _make_segsum(D, False)
_segsum16 = _make_segsum(16, False)


def _dotT(a, w):
  return lax.dot_general(a, w, (((1,), (1,)), ((), ())),
                         preferred_element_type=jnp.float32)


def _tc1_body(x_ref, agg_ref, cntp_ref, w1l_ref, b1l_ref, w1r_ref, out_ref):
  aggs = agg_ref[0] + agg_ref[1]
  cnt = cntp_ref[0, :, 0:1] + cntp_ref[1, :, 0:1]
  mean = aggs / jnp.maximum(cnt, 1.0)
  node1 = _dotT(mean, w1l_ref[...]) + b1l_ref[...] + _dotT(x_ref[...], w1r_ref[...])
  out_ref[...] = jnp.maximum(node1, 0.0)


def _tc1(x, agg, cntp, W1l, b1l, W1r):
  return pl.pallas_call(
      _tc1_body,
      out_shape=jax.ShapeDtypeStruct((N, D), jnp.float32),
  )(x, agg, cntp, W1l, b1l.reshape(1, D), W1r)


def _tc2_body(node1_ref, agg_ref, cntp_ref, w2l_ref, b2l_ref, w2r_ref,
              wfc1_ref, bfc1_ref, wfc2_ref, bfc2_ref,
              wc1_ref, bc1_ref, wc2_ref, bc2_ref,
              u_ref, noise_ref, label_ref,
              asg16_ref, graphf_ref, noisyg_ref, kl_ref, cls_ref):
  node1 = node1_ref[...]
  aggs = agg_ref[0] + agg_ref[1]
  cnt = cntp_ref[0, :, 0:1] + cntp_ref[1, :, 0:1]
  mean = aggs / jnp.maximum(cnt, 1.0)
  node2 = _dotT(mean, w2l_ref[...]) + b2l_ref[...] + _dotT(node1, w2r_ref[...])

  nf_mean = jnp.mean(node2, axis=0, keepdims=True)
  diff = node2 - nf_mean
  nf_var = jnp.sum(diff * diff, axis=0, keepdims=True) / (N - 1)
  nf_std = jnp.sqrt(nf_var)

  abstract1 = jnp.tanh(_dotT(node2, wfc1_ref[...]) + bfc1_ref[...])
  logits = _dotT(abstract1, wfc2_ref[...]) + bfc2_ref[...]
  m = jnp.max(logits, axis=1, keepdims=True)
  e = jnp.exp(logits - m)
  asg = e / jnp.sum(e, axis=1, keepdims=True)

  g = -jnp.log(-jnp.log(u_ref[...]))
  ga_l = asg + g
  gm = jnp.max(ga_l, axis=1, keepdims=True)
  ge = jnp.exp(ga_l - gm)
  ga = ge / jnp.sum(ge, axis=1, keepdims=True)
  lp = ga[:, 0:1]
  ln = ga[:, 1:2]

  graphf = jnp.sum(node2, axis=0, keepdims=True)
  graphf_ref[...] = graphf

  noisy_mean = lp * node2 + ln * nf_mean
  noisy_std = ln * nf_std
  noisy_node = noisy_mean + noise_ref[...] * noisy_std
  noisyg = jnp.sum(noisy_node, axis=0, keepdims=True)
  noisyg_ref[...] = noisyg

  denom = (nf_std + EPS) ** 2
  t1 = 0.5 * (noisy_std * noisy_std) / denom
  dmean = noisy_mean - nf_mean
  t2 = jnp.sum(dmean * dmean / denom, axis=0, keepdims=True)
  kl = jnp.sum(t1) / (N * D) + jnp.sum(t2) / D
  kl_ref[...] = jnp.reshape(kl, (1, 1))

  def classify(v):
    h = jnp.maximum(_dotT(v, wc1_ref[...]) + bc1_ref[...], 0.0)
    return jnp.maximum(_dotT(h, wc2_ref[...]) + bc2_ref[...], 0.0)

  lab = label_ref[...]
  c1 = classify(graphf) - lab
  c2 = classify(noisyg) - lab
  cls_ref[...] = c1 * c1 + c2 * c2

  col = lax.broadcasted_iota(jnp.int32, (N, 16), 1)
  asg16_ref[...] = jnp.where(col == 0, asg[:, 0:1],
                             jnp.where(col == 1, asg[:, 1:2], 0.0))


def _tc2(node1, agg, cntp, W2l, b2l, W2r, Wfc1, bfc1, Wfc2, bfc2,
         Wc1, bc1, Wc2, bc2, u, noise, label):
  return pl.pallas_call(
      _tc2_body,
      out_shape=(
          jax.ShapeDtypeStruct((N, 16), jnp.float32),
          jax.ShapeDtypeStruct((1, D), jnp.float32),
          jax.ShapeDtypeStruct((1, D), jnp.float32),
          jax.ShapeDtypeStruct((1, 1), jnp.float32),
          jax.ShapeDtypeStruct((1, 1), jnp.float32),
      ),
  )(node1, agg, cntp, W2l, b2l.reshape(1, D), W2r,
    Wfc1, bfc1.reshape(1, -1), Wfc2, bfc2.reshape(1, -1),
    Wc1, bc1.reshape(1, -1), Wc2, bc2.reshape(1, 1),
    u, noise, label.reshape(1, 1))


def _tc3_body(asg16_ref, bp_ref, out_ref):
  bsum = bp_ref[0] + bp_ref[1]
  madj = lax.dot_general(asg16_ref[...], bsum, (((0,), (0,)), ((), ())),
                         preferred_element_type=jnp.float32)
  na = madj[0:2, 0:2]
  denom = jnp.maximum(jnp.sum(jnp.abs(na), axis=1, keepdims=True), 1e-12)
  nrm = na / denom
  d0 = nrm[0:1, 0:1] - 1.0
  d1 = nrm[1:2, 1:2] - 1.0
  out_ref[...] = (d0 * d0 + d1 * d1) * 0.5


def _tc3(asg16, bp):
  return pl.pallas_call(
      _tc3_body,
      out_shape=jax.ShapeDtypeStruct((1, 1), jnp.float32),
  )(asg16, bp)


@jax.jit
def kernel(features, edges, label, W1l, b1l, W1r, W2l, b2l, W2r,
           Wfc1, bfc1, Wfc2, bfc2, Wc1, bc1, Wc2, bc2):
  src = edges[0]
  dst = edges[1]
  u = jax.random.uniform(jax.random.key(42), (N, 2), minval=1e-10, maxval=1.0)
  noise = jax.random.uniform(jax.random.key(43), (N, D))

  agg1, cntp = _segsum128_cnt(features, src, dst)
  node1 = _tc1(features, agg1, cntp, W1l, b1l, W1r)
  (agg2,) = _segsum128(node1, src, dst)
  asg16, graphf, noisyg, kl, cls = _tc2(
      node1, agg2, cntp, W2l, b2l, W2r, Wfc1, bfc1, Wfc2, bfc2,
      Wc1, bc1, Wc2, bc2, u, noise, label)
  (bp,) = _segsum16(asg16, dst, src)
  pp = _tc3(asg16, bp)

  return (graphf, noisyg, noisyg, kl[0, 0], cls[0, 0], pp[0, 0])


# speedup vs baseline: 4.3792x; 4.3792x over previous
import functools

import jax
import jax.numpy as jnp
from jax import lax
from jax.experimental import pallas as pl
from jax.experimental.pallas import tpu as pltpu
from jax.experimental.pallas import tpu_sc as plsc

N = 10000
E = 320000
D = 128
EPS = 1e-07

NC = 2
NS = 16
NW = NC * NS
EPW = E // NW
CHUNK = 80
NCHUNK = EPW // CHUNK
NP = 10240
RPS = NP // NS
NSLAB = RPS // CHUNK


def _zero_buf(ref, rows, width):
  def body(i, _):
    for j in range(width // 16):
      ref[i, pl.ds(j * 16, 16)] = jnp.zeros((16,), jnp.float32)
    return 0
  lax.fori_loop(0, rows, body, 0)


def _fill_ones(ref, rows, width):
  def body(i, _):
    for j in range(width // 16):
      ref[i, pl.ds(j * 16, 16)] = jnp.ones((16,), jnp.float32)
    return 0
  lax.fori_loop(0, rows, body, 0)


@functools.lru_cache(maxsize=None)
def _make_segsum(width):
  mesh = plsc.VectorSubcoreMesh(core_axis_name="c", subcore_axis_name="s",
                                num_cores=NC, num_subcores=NS)
  out_type = (jax.ShapeDtypeStruct((NC, NP, width), jnp.float32),)
  scratch = [
      pltpu.VMEM_SHARED((NP, width), jnp.float32),
      pltpu.VMEM((CHUNK,), jnp.int32),
      pltpu.VMEM((CHUNK,), jnp.int32),
      pltpu.VMEM((CHUNK, width), jnp.float32),
      pltpu.SemaphoreType.DMA,
  ]

  def body(table_hbm, gidx_hbm, sidx_hbm, out_hbm, acc_sh, gi_v, si_v, rows_v, sem):
    c = lax.axis_index("c")
    s = lax.axis_index("s")
    wid = c * NS + s

    _zero_buf(rows_v, CHUNK, width)
    row0 = s * RPS
    for r in range(NSLAB):
      pltpu.sync_copy(rows_v, acc_sh.at[pl.ds(row0 + r * CHUNK, CHUNK)])
    plsc.subcore_barrier()

    base = wid * EPW

    def chunk_body(i, _):
      off = pl.multiple_of(base + i * CHUNK, 8)
      pltpu.sync_copy(gidx_hbm.at[pl.ds(off, CHUNK)], gi_v)
      pltpu.sync_copy(sidx_hbm.at[pl.ds(off, CHUNK)], si_v)
      pltpu.async_copy(table_hbm.at[gi_v], rows_v, sem).wait()
      pltpu.sync_copy(rows_v, acc_sh.at[si_v], add=True)
      return 0

    lax.fori_loop(0, NCHUNK, chunk_body, 0)
    plsc.subcore_barrier()

    for r in range(NSLAB):
      rr = row0 + r * CHUNK
      pltpu.sync_copy(acc_sh.at[pl.ds(rr, CHUNK)], rows_v)
      pltpu.sync_copy(rows_v, out_hbm.at[c, pl.ds(rr, CHUNK)])

  return pl.kernel(body, out_type=out_type, mesh=mesh, scratch_types=scratch)


@functools.lru_cache(maxsize=None)
def _make_count():
  mesh = plsc.VectorSubcoreMesh(core_axis_name="c", subcore_axis_name="s",
                                num_cores=NC, num_subcores=NS)
  out_type = (jax.ShapeDtypeStruct((NC, NP, D), jnp.float32),)
  scratch = [
      pltpu.VMEM_SHARED((NP, D), jnp.float32),
      pltpu.VMEM((CHUNK,), jnp.int32),
      pltpu.VMEM((CHUNK, D), jnp.float32),
  ]

  def body(sidx_hbm, out_hbm, acc_sh, si_v, buf_v):
    c = lax.axis_index("c")
    s = lax.axis_index("s")
    wid = c * NS + s

    _zero_buf(buf_v, CHUNK, D)
    row0 = s * RPS
    for r in range(NSLAB):
      pltpu.sync_copy(buf_v, acc_sh.at[pl.ds(row0 + r * CHUNK, CHUNK)])
    _fill_ones(buf_v, CHUNK, D)
    plsc.subcore_barrier()

    base = wid * EPW

    def chunk_body(i, _):
      off = pl.multiple_of(base + i * CHUNK, 8)
      pltpu.sync_copy(sidx_hbm.at[pl.ds(off, CHUNK)], si_v)
      pltpu.sync_copy(buf_v, acc_sh.at[si_v], add=True)
      return 0

    lax.fori_loop(0, NCHUNK, chunk_body, 0)
    plsc.subcore_barrier()

    for r in range(NSLAB):
      rr = row0 + r * CHUNK
      pltpu.sync_copy(acc_sh.at[pl.ds(rr, CHUNK)], buf_v)
      pltpu.sync_copy(buf_v, out_hbm.at[c, pl.ds(rr, CHUNK)])

  return pl.kernel(body, out_type=out_type, mesh=mesh, scratch_types=scratch)


def _dotT(a, w):
  return lax.dot_general(a, w, (((1,), (1,)), ((), ())),
                         preferred_element_type=jnp.float32)


def _tc1_body(x_ref, agg_ref, cntp_ref, w1l_ref, b1l_ref, w1r_ref, out_ref):
  aggs = agg_ref[0, 0:N, :] + agg_ref[1, 0:N, :]
  cnt = cntp_ref[0, 0:N, 0:1] + cntp_ref[1, 0:N, 0:1]
  mean = aggs / jnp.maximum(cnt, 1.0)
  node1 = _dotT(mean, w1l_ref[...]) + b1l_ref[...] + _dotT(x_ref[...], w1r_ref[...])
  out_ref[...] = jnp.maximum(node1, 0.0)


def _tc1(x, agg, cntp, W1l, b1l, W1r):
  return pl.pallas_call(
      _tc1_body,
      out_shape=jax.ShapeDtypeStruct((N, D), jnp.float32),
  )(x, agg, cntp, W1l, b1l.reshape(1, D), W1r)


def _tc2a_body(node1_ref, agg_ref, cntp_ref, w2l_ref, b2l_ref, w2r_ref,
               node2_ref, graphf_ref, nfm_ref, nfs_ref):
  node1 = node1_ref[...]
  aggs = agg_ref[0, 0:N, :] + agg_ref[1, 0:N, :]
  cnt = cntp_ref[0, 0:N, 0:1] + cntp_ref[1, 0:N, 0:1]
  mean = aggs / jnp.maximum(cnt, 1.0)
  node2 = _dotT(mean, w2l_ref[...]) + b2l_ref[...] + _dotT(node1, w2r_ref[...])
  node2_ref[...] = node2

  nf_mean = jnp.mean(node2, axis=0, keepdims=True)
  diff = node2 - nf_mean
  nf_var = jnp.sum(diff * diff, axis=0, keepdims=True) / (N - 1)
  nfm_ref[...] = nf_mean
  nfs_ref[...] = jnp.sqrt(nf_var)
  graphf_ref[...] = jnp.sum(node2, axis=0, keepdims=True)


def _tc2a(node1, agg, cntp, W2l, b2l, W2r):
  return pl.pallas_call(
      _tc2a_body,
      out_shape=(
          jax.ShapeDtypeStruct((N, D), jnp.float32),
          jax.ShapeDtypeStruct((1, D), jnp.float32),
          jax.ShapeDtypeStruct((1, D), jnp.float32),
          jax.ShapeDtypeStruct((1, D), jnp.float32),
      ),
  )(node1, agg, cntp, W2l, b2l.reshape(1, D), W2r)


def _tcasg_body(node2_ref, wfc1_ref, bfc1_ref, wfc2_ref, bfc2_ref, u_ref,
                asg16_ref, lp_ref, ln_ref):
  node2 = node2_ref[...]
  abstract1 = jnp.tanh(_dotT(node2, wfc1_ref[...]) + bfc1_ref[...])
  logits = _dotT(abstract1, wfc2_ref[...]) + bfc2_ref[...]
  l0 = logits[:, 0:1]
  l1 = logits[:, 1:2]
  m = jnp.maximum(l0, l1)
  e0 = jnp.exp(l0 - m)
  e1 = jnp.exp(l1 - m)
  esum = e0 + e1
  a0 = e0 / esum
  a1 = e1 / esum

  g0 = -jnp.log(-jnp.log(u_ref[:, 0:1]))
  g1 = -jnp.log(-jnp.log(u_ref[:, 1:2]))
  q0 = a0 + g0
  q1 = a1 + g1
  gm = jnp.maximum(q0, q1)
  ge0 = jnp.exp(q0 - gm)
  ge1 = jnp.exp(q1 - gm)
  gsum = ge0 + ge1
  lp_ref[...] = ge0 / gsum
  ln_ref[...] = ge1 / gsum

  col = lax.broadcasted_iota(jnp.int32, (N, D), 1)
  asg16_ref[...] = jnp.where(col == 0, a0, jnp.where(col == 1, a1, 0.0))


def _tcasg(node2, Wfc1, bfc1, Wfc2, bfc2, u):
  return pl.pallas_call(
      _tcasg_body,
      out_shape=(
          jax.ShapeDtypeStruct((N, D), jnp.float32),
          jax.ShapeDtypeStruct((N, 1), jnp.float32),
          jax.ShapeDtypeStruct((N, 1), jnp.float32),
      ),
  )(node2, Wfc1, bfc1.reshape(1, -1),
    jnp.concatenate([Wfc2, jnp.zeros((6, Wfc2.shape[1]), jnp.float32)], axis=0),
    jnp.concatenate([bfc2, jnp.zeros((6,), jnp.float32)]).reshape(1, 8), u)


def _tc2b_body(node2_ref, lp_ref, ln_ref, graphf_ref, nfm_ref, nfs_ref,
               noise_ref, wc1_ref, bc1_ref, wc2_ref, bc2_ref, label_ref,
               noisyg_ref, kl_ref, cls_ref):
  node2 = node2_ref[...]
  lp = lp_ref[...]
  ln = ln_ref[...]
  nf_mean = nfm_ref[...]
  nf_std = nfs_ref[...]

  noisy_mean = lp * node2 + ln * nf_mean
  noisy_std = ln * nf_std
  noisy_node = noisy_mean + noise_ref[...] * noisy_std
  noisyg = jnp.sum(noisy_node, axis=0, keepdims=True)
  noisyg_ref[...] = noisyg

  denom = (nf_std + EPS) ** 2
  t1 = 0.5 * (noisy_std * noisy_std) / denom
  dmean = noisy_mean - nf_mean
  t2 = jnp.sum(dmean * dmean / denom, axis=0, keepdims=True)
  kl = jnp.sum(t1) / (N * D) + jnp.sum(t2) / D
  kl_ref[...] = kl * jnp.ones((1, 1), jnp.float32)

  v2 = jnp.concatenate([graphf_ref[...], noisyg], axis=0)
  h = jnp.maximum(_dotT(v2, wc1_ref[...]) + bc1_ref[...], 0.0)
  o = jnp.sum(h * wc2_ref[...], axis=1, keepdims=True) + bc2_ref[...]
  o = jnp.maximum(o, 0.0)
  cd = o - label_ref[...]
  cls_ref[...] = jnp.sum(cd * cd, axis=0, keepdims=True)


def _tc2b(node2, lp, ln, graphf, nfm, nfs, noise, Wc1, bc1, Wc2, bc2, label):
  return pl.pallas_call(
      _tc2b_body,
      out_shape=(
          jax.ShapeDtypeStruct((1, D), jnp.float32),
          jax.ShapeDtypeStruct((1, 1), jnp.float32),
          jax.ShapeDtypeStruct((1, 1), jnp.float32),
      ),
  )(node2, lp, ln, graphf, nfm, nfs, noise,
    Wc1, bc1.reshape(1, -1), Wc2, bc2.reshape(1, 1),
    jnp.broadcast_to(label.reshape(1, 1), (2, 1)))


def _tc3_body(asg16_ref, bp_ref, out_ref):
  bsum = bp_ref[0, 0:N, :] + bp_ref[1, 0:N, :]
  madj = lax.dot_general(asg16_ref[...], bsum, (((0,), (0,)), ((), ())),
                         preferred_element_type=jnp.float32)
  na = madj[0:2, :]
  denom = jnp.maximum(jnp.sum(jnp.abs(na), axis=1, keepdims=True), 1e-12)
  row = lax.broadcasted_iota(jnp.int32, (2, D), 0)
  col = lax.broadcasted_iota(jnp.int32, (2, D), 1)
  diag = jnp.sum(jnp.where(row == col, na, 0.0), axis=1, keepdims=True)
  nd = diag / denom - 1.0
  out_ref[...] = jnp.sum(nd * nd, axis=0, keepdims=True) * 0.5


def _tc3(asg16, bp):
  return pl.pallas_call(
      _tc3_body,
      out_shape=jax.ShapeDtypeStruct((1, 1), jnp.float32),
  )(asg16, bp)


@jax.jit
def kernel(features, edges, label, W1l, b1l, W1r, W2l, b2l, W2r,
           Wfc1, bfc1, Wfc2, bfc2, Wc1, bc1, Wc2, bc2):
  src = edges[0]
  dst = edges[1]
  u = jax.random.uniform(jax.random.key(42), (N, 2), minval=1e-10, maxval=1.0)
  noise = jax.random.uniform(jax.random.key(43), (N, D))

  (cnt128,) = _make_count()(dst)
  cntp = cnt128[:, :, 0:16]
  (agg1,) = _make_segsum(D)(features, src, dst)
  node1 = _tc1(features, agg1, cntp, W1l, b1l, W1r)
  (agg2,) = _make_segsum(D)(node1, src, dst)
  node2, graphf, nfm, nfs = _tc2a(node1, agg2, cntp, W2l, b2l, W2r)
  asg16, lp, ln = _tcasg(node2, Wfc1, bfc1, Wfc2, bfc2, u)
  noisyg, kl, cls = _tc2b(node2, lp, ln, graphf, nfm, nfs, noise,
                          Wc1, bc1, Wc2, bc2, label)
  (bp,) = _make_segsum(D)(asg16, dst, src)
  pp = _tc3(asg16, bp)

  return (graphf, noisyg, noisyg, kl[0, 0], cls[0, 0], pp[0, 0])


# speedup vs baseline: 7.9088x; 1.8060x over previous
import functools

import jax
import jax.numpy as jnp
from jax import lax
from jax.experimental import pallas as pl
from jax.experimental.pallas import tpu as pltpu
from jax.experimental.pallas import tpu_sc as plsc

N = 10000
E = 320000
D = 128
EPS = 1e-07

NC = 2
NS = 16
NW = NC * NS
EPW = E // NW
CHUNK = 80
NCHUNK = EPW // CHUNK
NP = 10240
RPS = NP // NS
NSLAB = RPS // CHUNK
NSLOT = 4


def _zero_buf(ref, rows, width):
  def body(i, _):
    for j in range(width // 16):
      ref[i, pl.ds(j * 16, 16)] = jnp.zeros((16,), jnp.float32)
    return 0
  lax.fori_loop(0, rows, body, 0)


def _fill_ones(ref, rows, width):
  def body(i, _):
    for j in range(width // 16):
      ref[i, pl.ds(j * 16, 16)] = jnp.ones((16,), jnp.float32)
    return 0
  lax.fori_loop(0, rows, body, 0)


@functools.lru_cache(maxsize=None)
def _make_segsum(width):
  mesh = plsc.VectorSubcoreMesh(core_axis_name="c", subcore_axis_name="s",
                                num_cores=NC, num_subcores=NS)
  out_type = (jax.ShapeDtypeStruct((NC, NP, width), jnp.float32),)
  scratch = (
      [pltpu.VMEM_SHARED((NP, width), jnp.float32)]
      + [pltpu.VMEM((CHUNK,), jnp.int32)] * NSLOT
      + [pltpu.VMEM((CHUNK,), jnp.int32)] * NSLOT
      + [pltpu.VMEM((CHUNK, width), jnp.float32)] * NSLOT
      + [pltpu.SemaphoreType.DMA] * (3 * NSLOT)
  )

  def body(table_hbm, gidx_hbm, sidx_hbm, out_hbm, acc_sh, *rest):
    gi_v = rest[0:NSLOT]
    si_v = rest[NSLOT:2 * NSLOT]
    rows_v = rest[2 * NSLOT:3 * NSLOT]
    isem = rest[3 * NSLOT:4 * NSLOT]
    gsem = rest[4 * NSLOT:5 * NSLOT]
    ssem = rest[5 * NSLOT:6 * NSLOT]
    c = lax.axis_index("c")
    s = lax.axis_index("s")
    wid = c * NS + s

    _zero_buf(rows_v[0], CHUNK, width)
    row0 = s * RPS
    for r in range(NSLAB):
      pltpu.sync_copy(rows_v[0], acc_sh.at[pl.ds(row0 + r * CHUNK, CHUNK)])
    plsc.subcore_barrier()

    base = wid * EPW

    def quad_body(i, _):
      c0 = i * NSLOT
      idescs = []
      for k in range(NSLOT):
        off = pl.multiple_of(base + (c0 + k) * CHUNK, 8)
        d1 = pltpu.async_copy(gidx_hbm.at[pl.ds(off, CHUNK)], gi_v[k], isem[k])
        d2 = pltpu.async_copy(sidx_hbm.at[pl.ds(off, CHUNK)], si_v[k], isem[k])
        idescs.append((d1, d2))
      gdescs = []
      for k in range(NSLOT):
        idescs[k][0].wait()
        idescs[k][1].wait()
        gdescs.append(pltpu.async_copy(table_hbm.at[gi_v[k]], rows_v[k], gsem[k]))
      sdescs = []
      for k in range(NSLOT):
        gdescs[k].wait()
        sdescs.append(pltpu.async_copy(rows_v[k], acc_sh.at[si_v[k]], ssem[k],
                                       add=True))
      for k in range(NSLOT):
        sdescs[k].wait()
      return 0

    lax.fori_loop(0, NCHUNK // NSLOT, quad_body, 0)
    for t in range((NCHUNK // NSLOT) * NSLOT, NCHUNK):
      off = pl.multiple_of(base + t * CHUNK, 8)
      pltpu.sync_copy(gidx_hbm.at[pl.ds(off, CHUNK)], gi_v[0])
      pltpu.sync_copy(sidx_hbm.at[pl.ds(off, CHUNK)], si_v[0])
      pltpu.async_copy(table_hbm.at[gi_v[0]], rows_v[0], gsem[0]).wait()
      pltpu.sync_copy(rows_v[0], acc_sh.at[si_v[0]], add=True)
    plsc.subcore_barrier()

    for r in range(NSLAB):
      rr = row0 + r * CHUNK
      b = rows_v[r % NSLOT]
      pltpu.sync_copy(acc_sh.at[pl.ds(rr, CHUNK)], b)
      pltpu.sync_copy(b, out_hbm.at[c, pl.ds(rr, CHUNK)])

  return pl.kernel(body, out_type=out_type, mesh=mesh, scratch_types=scratch)


@functools.lru_cache(maxsize=None)
def _make_count():
  mesh = plsc.VectorSubcoreMesh(core_axis_name="c", subcore_axis_name="s",
                                num_cores=NC, num_subcores=NS)
  out_type = (jax.ShapeDtypeStruct((NC, NP, D), jnp.float32),)
  scratch = (
      [pltpu.VMEM_SHARED((NP, D), jnp.float32)]
      + [pltpu.VMEM((CHUNK,), jnp.int32)] * NSLOT
      + [pltpu.VMEM((CHUNK, D), jnp.float32)]
      + [pltpu.SemaphoreType.DMA] * (2 * NSLOT)
  )

  def body(sidx_hbm, out_hbm, acc_sh, *rest):
    si_v = rest[0:NSLOT]
    buf_v = rest[NSLOT]
    isem = rest[NSLOT + 1:NSLOT + 1 + NSLOT]
    ssem = rest[NSLOT + 1 + NSLOT:]
    c = lax.axis_index("c")
    s = lax.axis_index("s")
    wid = c * NS + s

    _zero_buf(buf_v, CHUNK, D)
    row0 = s * RPS
    for r in range(NSLAB):
      pltpu.sync_copy(buf_v, acc_sh.at[pl.ds(row0 + r * CHUNK, CHUNK)])
    _fill_ones(buf_v, CHUNK, D)
    plsc.subcore_barrier()

    base = wid * EPW

    def quad_body(i, _):
      c0 = i * NSLOT
      idescs = []
      for k in range(NSLOT):
        off = pl.multiple_of(base + (c0 + k) * CHUNK, 8)
        idescs.append(
            pltpu.async_copy(sidx_hbm.at[pl.ds(off, CHUNK)], si_v[k], isem[k]))
      sdescs = []
      for k in range(NSLOT):
        idescs[k].wait()
        sdescs.append(pltpu.async_copy(buf_v, acc_sh.at[si_v[k]], ssem[k],
                                       add=True))
      for k in range(NSLOT):
        sdescs[k].wait()
      return 0

    lax.fori_loop(0, NCHUNK // NSLOT, quad_body, 0)
    for t in range((NCHUNK // NSLOT) * NSLOT, NCHUNK):
      off = pl.multiple_of(base + t * CHUNK, 8)
      pltpu.sync_copy(sidx_hbm.at[pl.ds(off, CHUNK)], si_v[0])
      pltpu.sync_copy(buf_v, acc_sh.at[si_v[0]], add=True)
    plsc.subcore_barrier()

    for r in range(NSLAB):
      rr = row0 + r * CHUNK
      pltpu.sync_copy(acc_sh.at[pl.ds(rr, CHUNK)], buf_v)
      pltpu.sync_copy(buf_v, out_hbm.at[c, pl.ds(rr, CHUNK)])

  return pl.kernel(body, out_type=out_type, mesh=mesh, scratch_types=scratch)


def _dotT(a, w):
  return lax.dot_general(a, w, (((1,), (1,)), ((), ())),
                         preferred_element_type=jnp.float32)


def _tc1_body(x_ref, agg_ref, cntp_ref, w1l_ref, b1l_ref, w1r_ref, out_ref):
  aggs = agg_ref[0, 0:N, :] + agg_ref[1, 0:N, :]
  cnt = cntp_ref[0, 0:N, 0:1] + cntp_ref[1, 0:N, 0:1]
  mean = aggs / jnp.maximum(cnt, 1.0)
  node1 = _dotT(mean, w1l_ref[...]) + b1l_ref[...] + _dotT(x_ref[...], w1r_ref[...])
  out_ref[...] = jnp.maximum(node1, 0.0)


def _tc1(x, agg, cntp, W1l, b1l, W1r):
  return pl.pallas_call(
      _tc1_body,
      out_shape=jax.ShapeDtypeStruct((N, D), jnp.float32),
  )(x, agg, cntp, W1l, b1l.reshape(1, D), W1r)


def _tc2a_body(node1_ref, agg_ref, cntp_ref, w2l_ref, b2l_ref, w2r_ref,
               node2_ref, graphf_ref, nfm_ref, nfs_ref):
  node1 = node1_ref[...]
  aggs = agg_ref[0, 0:N, :] + agg_ref[1, 0:N, :]
  cnt = cntp_ref[0, 0:N, 0:1] + cntp_ref[1, 0:N, 0:1]
  mean = aggs / jnp.maximum(cnt, 1.0)
  node2 = _dotT(mean, w2l_ref[...]) + b2l_ref[...] + _dotT(node1, w2r_ref[...])
  node2_ref[...] = node2

  nf_mean = jnp.mean(node2, axis=0, keepdims=True)
  diff = node2 - nf_mean
  nf_var = jnp.sum(diff * diff, axis=0, keepdims=True) / (N - 1)
  nfm_ref[...] = nf_mean
  nfs_ref[...] = jnp.sqrt(nf_var)
  graphf_ref[...] = jnp.sum(node2, axis=0, keepdims=True)


def _tc2a(node1, agg, cntp, W2l, b2l, W2r):
  return pl.pallas_call(
      _tc2a_body,
      out_shape=(
          jax.ShapeDtypeStruct((N, D), jnp.float32),
          jax.ShapeDtypeStruct((1, D), jnp.float32),
          jax.ShapeDtypeStruct((1, D), jnp.float32),
          jax.ShapeDtypeStruct((1, D), jnp.float32),
      ),
  )(node1, agg, cntp, W2l, b2l.reshape(1, D), W2r)


def _tcasg_body(node2_ref, wfc1_ref, bfc1_ref, wfc2_ref, bfc2_ref, u_ref,
                asg16_ref, lp_ref, ln_ref):
  node2 = node2_ref[...]
  abstract1 = jnp.tanh(_dotT(node2, wfc1_ref[...]) + bfc1_ref[...])
  logits = _dotT(abstract1, wfc2_ref[...]) + bfc2_ref[...]
  l0 = logits[:, 0:1]
  l1 = logits[:, 1:2]
  m = jnp.maximum(l0, l1)
  e0 = jnp.exp(l0 - m)
  e1 = jnp.exp(l1 - m)
  esum = e0 + e1
  a0 = e0 / esum
  a1 = e1 / esum

  g0 = -jnp.log(-jnp.log(u_ref[:, 0:1]))
  g1 = -jnp.log(-jnp.log(u_ref[:, 1:2]))
  q0 = a0 + g0
  q1 = a1 + g1
  gm = jnp.maximum(q0, q1)
  ge0 = jnp.exp(q0 - gm)
  ge1 = jnp.exp(q1 - gm)
  gsum = ge0 + ge1
  lp_ref[...] = ge0 / gsum
  ln_ref[...] = ge1 / gsum

  col = lax.broadcasted_iota(jnp.int32, (N, D), 1)
  asg16_ref[...] = jnp.where(col == 0, a0, jnp.where(col == 1, a1, 0.0))


def _tcasg(node2, Wfc1, bfc1, Wfc2, bfc2, u):
  return pl.pallas_call(
      _tcasg_body,
      out_shape=(
          jax.ShapeDtypeStruct((N, D), jnp.float32),
          jax.ShapeDtypeStruct((N, 1), jnp.float32),
          jax.ShapeDtypeStruct((N, 1), jnp.float32),
      ),
  )(node2, Wfc1, bfc1.reshape(1, -1),
    jnp.concatenate([Wfc2, jnp.zeros((6, Wfc2.shape[1]), jnp.float32)], axis=0),
    jnp.concatenate([bfc2, jnp.zeros((6,), jnp.float32)]).reshape(1, 8), u)


def _tc2b_body(node2_ref, lp_ref, ln_ref, graphf_ref, nfm_ref, nfs_ref,
               noise_ref, wc1_ref, bc1_ref, wc2_ref, bc2_ref, label_ref,
               noisyg_ref, kl_ref, cls_ref):
  node2 = node2_ref[...]
  lp = lp_ref[...]
  ln = ln_ref[...]
  nf_mean = nfm_ref[...]
  nf_std = nfs_ref[...]

  noisy_mean = lp * node2 + ln * nf_mean
  noisy_std = ln * nf_std
  noisy_node = noisy_mean + noise_ref[...] * noisy_std
  noisyg = jnp.sum(noisy_node, axis=0, keepdims=True)
  noisyg_ref[...] = noisyg

  denom = (nf_std + EPS) ** 2
  t1 = 0.5 * (noisy_std * noisy_std) / denom
  dmean = noisy_mean - nf_mean
  t2 = jnp.sum(dmean * dmean / denom, axis=0, keepdims=True)
  kl = jnp.sum(t1) / (N * D) + jnp.sum(t2) / D
  kl_ref[...] = kl * jnp.ones((1, 1), jnp.float32)

  v2 = jnp.concatenate([graphf_ref[...], noisyg], axis=0)
  h = jnp.maximum(_dotT(v2, wc1_ref[...]) + bc1_ref[...], 0.0)
  o = jnp.sum(h * wc2_ref[...], axis=1, keepdims=True) + bc2_ref[...]
  o = jnp.maximum(o, 0.0)
  cd = o - label_ref[...]
  cls_ref[...] = jnp.sum(cd * cd, axis=0, keepdims=True)


def _tc2b(node2, lp, ln, graphf, nfm, nfs, noise, Wc1, bc1, Wc2, bc2, label):
  return pl.pallas_call(
      _tc2b_body,
      out_shape=(
          jax.ShapeDtypeStruct((1, D), jnp.float32),
          jax.ShapeDtypeStruct((1, 1), jnp.float32),
          jax.ShapeDtypeStruct((1, 1), jnp.float32),
      ),
  )(node2, lp, ln, graphf, nfm, nfs, noise,
    Wc1, bc1.reshape(1, -1), Wc2, bc2.reshape(1, 1),
    jnp.broadcast_to(label.reshape(1, 1), (2, 1)))


def _tc3_body(asg16_ref, bp_ref, out_ref):
  bsum = bp_ref[0, 0:N, :] + bp_ref[1, 0:N, :]
  madj = lax.dot_general(asg16_ref[...], bsum, (((0,), (0,)), ((), ())),
                         preferred_element_type=jnp.float32)
  na = madj[0:2, :]
  denom = jnp.maximum(jnp.sum(jnp.abs(na), axis=1, keepdims=True), 1e-12)
  row = lax.broadcasted_iota(jnp.int32, (2, D), 0)
  col = lax.broadcasted_iota(jnp.int32, (2, D), 1)
  diag = jnp.sum(jnp.where(row == col, na, 0.0), axis=1, keepdims=True)
  nd = diag / denom - 1.0
  out_ref[...] = jnp.sum(nd * nd, axis=0, keepdims=True) * 0.5


def _tc3(asg16, bp):
  return pl.pallas_call(
      _tc3_body,
      out_shape=jax.ShapeDtypeStruct((1, 1), jnp.float32),
  )(asg16, bp)


@jax.jit
def kernel(features, edges, label, W1l, b1l, W1r, W2l, b2l, W2r,
           Wfc1, bfc1, Wfc2, bfc2, Wc1, bc1, Wc2, bc2):
  src = edges[0]
  dst = edges[1]
  u = jax.random.uniform(jax.random.key(42), (N, 2), minval=1e-10, maxval=1.0)
  noise = jax.random.uniform(jax.random.key(43), (N, D))

  (cnt128,) = _make_count()(dst)
  cntp = cnt128[:, :, 0:16]
  (agg1,) = _make_segsum(D)(features, src, dst)
  node1 = _tc1(features, agg1, cntp, W1l, b1l, W1r)
  (agg2,) = _make_segsum(D)(node1, src, dst)
  node2, graphf, nfm, nfs = _tc2a(node1, agg2, cntp, W2l, b2l, W2r)
  asg16, lp, ln = _tcasg(node2, Wfc1, bfc1, Wfc2, bfc2, u)
  noisyg, kl, cls = _tc2b(node2, lp, ln, graphf, nfm, nfs, noise,
                          Wc1, bc1, Wc2, bc2, label)
  (bp,) = _make_segsum(D)(asg16, dst, src)
  pp = _tc3(asg16, bp)

  return (graphf, noisyg, noisyg, kl[0, 0], cls[0, 0], pp[0, 0])


# speedup vs baseline: 9.8640x; 1.2472x over previous
import functools

import jax
import jax.numpy as jnp
from jax import lax
from jax.experimental import pallas as pl
from jax.experimental.pallas import tpu as pltpu
from jax.experimental.pallas import tpu_sc as plsc

N = 10000
E = 320000
D = 128
EPS = 1e-07

NC = 2
NS = 16
NW = NC * NS
EPW = E // NW
CHUNK = 80
NCHUNK = EPW // CHUNK
NP = 10240
RPS = NP // NS
NSLAB = RPS // CHUNK
NSLOT = 4


def _zero_buf(ref, rows, width):
  def body(i, _):
    for j in range(width // 16):
      ref[i, pl.ds(j * 16, 16)] = jnp.zeros((16,), jnp.float32)
    return 0
  lax.fori_loop(0, rows, body, 0)


def _fill_ones(ref, rows, width):
  def body(i, _):
    for j in range(width // 16):
      ref[i, pl.ds(j * 16, 16)] = jnp.ones((16,), jnp.float32)
    return 0
  lax.fori_loop(0, rows, body, 0)


@functools.lru_cache(maxsize=None)
def _make_segsum(width):
  mesh = plsc.VectorSubcoreMesh(core_axis_name="c", subcore_axis_name="s",
                                num_cores=NC, num_subcores=NS)
  out_type = (jax.ShapeDtypeStruct((NC, NP, width), jnp.float32),)
  scratch = (
      [pltpu.VMEM_SHARED((NP, width), jnp.float32)]
      + [pltpu.VMEM((CHUNK,), jnp.int32)] * NSLOT
      + [pltpu.VMEM((CHUNK,), jnp.int32)] * NSLOT
      + [pltpu.VMEM((CHUNK, width), jnp.float32)] * NSLOT
      + [pltpu.SemaphoreType.DMA] * (3 * NSLOT)
  )

  def body(table_hbm, gidx_hbm, sidx_hbm, out_hbm, acc_sh, *rest):
    gi_v = rest[0:NSLOT]
    si_v = rest[NSLOT:2 * NSLOT]
    rows_v = rest[2 * NSLOT:3 * NSLOT]
    isem = rest[3 * NSLOT:4 * NSLOT]
    gsem = rest[4 * NSLOT:5 * NSLOT]
    ssem = rest[5 * NSLOT:6 * NSLOT]
    c = lax.axis_index("c")
    s = lax.axis_index("s")
    wid = c * NS + s

    _zero_buf(rows_v[0], CHUNK, width)
    row0 = s * RPS
    for r in range(NSLAB):
      pltpu.sync_copy(rows_v[0], acc_sh.at[pl.ds(row0 + r * CHUNK, CHUNK)])
    plsc.subcore_barrier()

    base = wid * EPW

    def quad_body(i, _):
      c0 = i * NSLOT
      idescs = []
      for k in range(NSLOT):
        off = pl.multiple_of(base + (c0 + k) * CHUNK, 8)
        d1 = pltpu.async_copy(gidx_hbm.at[pl.ds(off, CHUNK)], gi_v[k], isem[k])
        d2 = pltpu.async_copy(sidx_hbm.at[pl.ds(off, CHUNK)], si_v[k], isem[k])
        idescs.append((d1, d2))
      gdescs = []
      for k in range(NSLOT):
        idescs[k][0].wait()
        idescs[k][1].wait()
        gdescs.append(pltpu.async_copy(table_hbm.at[gi_v[k]], rows_v[k], gsem[k]))
      sdescs = []
      for k in range(NSLOT):
        gdescs[k].wait()
        sdescs.append(pltpu.async_copy(rows_v[k], acc_sh.at[si_v[k]], ssem[k],
                                       add=True))
      for k in range(NSLOT):
        sdescs[k].wait()
      return 0

    lax.fori_loop(0, NCHUNK // NSLOT, quad_body, 0)
    for t in range((NCHUNK // NSLOT) * NSLOT, NCHUNK):
      off = pl.multiple_of(base + t * CHUNK, 8)
      pltpu.sync_copy(gidx_hbm.at[pl.ds(off, CHUNK)], gi_v[0])
      pltpu.sync_copy(sidx_hbm.at[pl.ds(off, CHUNK)], si_v[0])
      pltpu.async_copy(table_hbm.at[gi_v[0]], rows_v[0], gsem[0]).wait()
      pltpu.sync_copy(rows_v[0], acc_sh.at[si_v[0]], add=True)
    plsc.subcore_barrier()

    for r in range(NSLAB):
      rr = row0 + r * CHUNK
      b = rows_v[r % NSLOT]
      pltpu.sync_copy(acc_sh.at[pl.ds(rr, CHUNK)], b)
      pltpu.sync_copy(b, out_hbm.at[c, pl.ds(rr, CHUNK)])

  return pl.kernel(body, out_type=out_type, mesh=mesh, scratch_types=scratch)


@functools.lru_cache(maxsize=None)
def _make_count():
  mesh = plsc.VectorSubcoreMesh(core_axis_name="c", subcore_axis_name="s",
                                num_cores=NC, num_subcores=NS)
  out_type = (jax.ShapeDtypeStruct((NC, NP, D), jnp.float32),)
  scratch = (
      [pltpu.VMEM_SHARED((NP, D), jnp.float32)]
      + [pltpu.VMEM((CHUNK,), jnp.int32)] * NSLOT
      + [pltpu.VMEM((CHUNK, D), jnp.float32)]
      + [pltpu.SemaphoreType.DMA] * (2 * NSLOT)
  )

  def body(sidx_hbm, out_hbm, acc_sh, *rest):
    si_v = rest[0:NSLOT]
    buf_v = rest[NSLOT]
    isem = rest[NSLOT + 1:NSLOT + 1 + NSLOT]
    ssem = rest[NSLOT + 1 + NSLOT:]
    c = lax.axis_index("c")
    s = lax.axis_index("s")
    wid = c * NS + s

    _zero_buf(buf_v, CHUNK, D)
    row0 = s * RPS
    for r in range(NSLAB):
      pltpu.sync_copy(buf_v, acc_sh.at[pl.ds(row0 + r * CHUNK, CHUNK)])
    _fill_ones(buf_v, CHUNK, D)
    plsc.subcore_barrier()

    base = wid * EPW

    def quad_body(i, _):
      c0 = i * NSLOT
      idescs = []
      for k in range(NSLOT):
        off = pl.multiple_of(base + (c0 + k) * CHUNK, 8)
        idescs.append(
            pltpu.async_copy(sidx_hbm.at[pl.ds(off, CHUNK)], si_v[k], isem[k]))
      sdescs = []
      for k in range(NSLOT):
        idescs[k].wait()
        sdescs.append(pltpu.async_copy(buf_v, acc_sh.at[si_v[k]], ssem[k],
                                       add=True))
      for k in range(NSLOT):
        sdescs[k].wait()
      return 0

    lax.fori_loop(0, NCHUNK // NSLOT, quad_body, 0)
    for t in range((NCHUNK // NSLOT) * NSLOT, NCHUNK):
      off = pl.multiple_of(base + t * CHUNK, 8)
      pltpu.sync_copy(sidx_hbm.at[pl.ds(off, CHUNK)], si_v[0])
      pltpu.sync_copy(buf_v, acc_sh.at[si_v[0]], add=True)
    plsc.subcore_barrier()

    for r in range(NSLAB):
      rr = row0 + r * CHUNK
      pltpu.sync_copy(acc_sh.at[pl.ds(rr, CHUNK)], buf_v)
      pltpu.sync_copy(buf_v, out_hbm.at[c, pl.ds(rr, CHUNK)])

  return pl.kernel(body, out_type=out_type, mesh=mesh, scratch_types=scratch)


@functools.lru_cache(maxsize=None)
def _make_adj():
  mesh = plsc.VectorSubcoreMesh(core_axis_name="c", subcore_axis_name="s",
                                num_cores=NC, num_subcores=NS)
  scratch = [
      pltpu.VMEM_SHARED((NP,), jnp.float32),
      pltpu.VMEM((NP,), jnp.float32),
      pltpu.VMEM((RPS,), jnp.float32),
      pltpu.VMEM((EPW,), jnp.int32),
      pltpu.VMEM((EPW,), jnp.int32),
      pltpu.VMEM((CHUNK, D), jnp.float32),
      pltpu.VMEM((16,), jnp.float32),
  ]

  def body(asgp_hbm, gidx_hbm, sidx_hbm, out_hbm,
           a0_sh, a0_v, slice_v, gi_all, si_all, slab_v, outbuf):
    c = lax.axis_index("c")
    s = lax.axis_index("s")
    wid = c * NS + s
    row0 = s * RPS
    zero16 = jnp.zeros((16,), jnp.int32)
    for r in range(RPS // CHUNK):
      pltpu.sync_copy(asgp_hbm.at[pl.ds(row0 + r * CHUNK, CHUNK)], slab_v)

      def ext(j, _, _r=r):
        idx = lax.broadcasted_iota(jnp.int32, (16,), 0) + j * 16
        v = plsc.load_gather(slab_v, [idx, zero16])
        plsc.store_scatter(slice_v, [idx + _r * CHUNK], v)
        return 0

      lax.fori_loop(0, CHUNK // 16, ext, 0)
    pltpu.sync_copy(slice_v, a0_sh.at[pl.ds(row0, RPS)])
    plsc.subcore_barrier()
    pltpu.sync_copy(a0_sh, a0_v)

    base = wid * EPW
    pltpu.sync_copy(gidx_hbm.at[pl.ds(base, EPW)], gi_all)
    pltpu.sync_copy(sidx_hbm.at[pl.ds(base, EPW)], si_all)

    def grp(j, carry):
      accT, accS, accD = carry
      gs = gi_all[pl.ds(j * 16, 16)]
      ss = si_all[pl.ds(j * 16, 16)]
      vs = plsc.load_gather(a0_v, [gs])
      vd = plsc.load_gather(a0_v, [ss])
      return (accT + vs * vd, accS + vs, accD + vd)

    z = jnp.zeros((16,), jnp.float32)
    accT, accS, accD = lax.fori_loop(0, EPW // 16, grp, (z, z, z))
    t = jnp.sum(accT)
    sg = jnp.sum(accS)
    sd = jnp.sum(accD)
    lane = lax.broadcasted_iota(jnp.int32, (16,), 0)
    outbuf[...] = jnp.where(lane == 0, t,
                            jnp.where(lane == 1, sg,
                                      jnp.where(lane == 2, sd, 0.0)))
    pltpu.sync_copy(outbuf, out_hbm.at[pl.ds(wid * 16, 16)])

  return pl.kernel(body, out_type=(jax.ShapeDtypeStruct((NW * 16,), jnp.float32),),
                   mesh=mesh, scratch_types=scratch,
                   compiler_params=pltpu.CompilerParams(needs_layout_passes=False))


def _dotT(a, w):
  return lax.dot_general(a, w, (((1,), (1,)), ((), ())),
                         preferred_element_type=jnp.float32)


def _tc1_body(x_ref, agg_ref, cntp_ref, w1l_ref, b1l_ref, w1r_ref, out_ref):
  aggs = agg_ref[0, 0:N, :] + agg_ref[1, 0:N, :]
  cnt = cntp_ref[0, 0:N, 0:1] + cntp_ref[1, 0:N, 0:1]
  mean = aggs / jnp.maximum(cnt, 1.0)
  node1 = _dotT(mean, w1l_ref[...]) + b1l_ref[...] + _dotT(x_ref[...], w1r_ref[...])
  out_ref[...] = jnp.maximum(node1, 0.0)


def _tc1(x, agg, cntp, W1l, b1l, W1r):
  return pl.pallas_call(
      _tc1_body,
      out_shape=jax.ShapeDtypeStruct((N, D), jnp.float32),
  )(x, agg, cntp, W1l, b1l.reshape(1, D), W1r)


def _tc2a_body(node1_ref, agg_ref, cntp_ref, w2l_ref, b2l_ref, w2r_ref,
               node2_ref, graphf_ref, nfm_ref, nfs_ref):
  node1 = node1_ref[...]
  aggs = agg_ref[0, 0:N, :] + agg_ref[1, 0:N, :]
  cnt = cntp_ref[0, 0:N, 0:1] + cntp_ref[1, 0:N, 0:1]
  mean = aggs / jnp.maximum(cnt, 1.0)
  node2 = _dotT(mean, w2l_ref[...]) + b2l_ref[...] + _dotT(node1, w2r_ref[...])
  node2_ref[...] = node2

  nf_mean = jnp.mean(node2, axis=0, keepdims=True)
  diff = node2 - nf_mean
  nf_var = jnp.sum(diff * diff, axis=0, keepdims=True) / (N - 1)
  nfm_ref[...] = nf_mean
  nfs_ref[...] = jnp.sqrt(nf_var)
  graphf_ref[...] = jnp.sum(node2, axis=0, keepdims=True)


def _tc2a(node1, agg, cntp, W2l, b2l, W2r):
  return pl.pallas_call(
      _tc2a_body,
      out_shape=(
          jax.ShapeDtypeStruct((N, D), jnp.float32),
          jax.ShapeDtypeStruct((1, D), jnp.float32),
          jax.ShapeDtypeStruct((1, D), jnp.float32),
          jax.ShapeDtypeStruct((1, D), jnp.float32),
      ),
  )(node1, agg, cntp, W2l, b2l.reshape(1, D), W2r)


def _tcasg_body(node2_ref, wfc1_ref, bfc1_ref, wfc2_ref, bfc2_ref, u_ref,
                asg16_ref, lp_ref, ln_ref):
  node2 = node2_ref[...]
  abstract1 = jnp.tanh(_dotT(node2, wfc1_ref[...]) + bfc1_ref[...])
  logits = _dotT(abstract1, wfc2_ref[...]) + bfc2_ref[...]
  l0 = logits[:, 0:1]
  l1 = logits[:, 1:2]
  m = jnp.maximum(l0, l1)
  e0 = jnp.exp(l0 - m)
  e1 = jnp.exp(l1 - m)
  esum = e0 + e1
  a0 = e0 / esum
  a1 = e1 / esum

  g0 = -jnp.log(-jnp.log(u_ref[:, 0:1]))
  g1 = -jnp.log(-jnp.log(u_ref[:, 1:2]))
  q0 = a0 + g0
  q1 = a1 + g1
  gm = jnp.maximum(q0, q1)
  ge0 = jnp.exp(q0 - gm)
  ge1 = jnp.exp(q1 - gm)
  gsum = ge0 + ge1
  lp_ref[...] = ge0 / gsum
  ln_ref[...] = ge1 / gsum

  col = lax.broadcasted_iota(jnp.int32, (N, D), 1)
  asg16_ref[0:N, :] = jnp.where(col == 0, a0, jnp.where(col == 1, a1, 0.0))
  asg16_ref[N:NP, :] = jnp.zeros((NP - N, D), jnp.float32)


def _tcasg(node2, Wfc1, bfc1, Wfc2, bfc2, u):
  return pl.pallas_call(
      _tcasg_body,
      out_shape=(
          jax.ShapeDtypeStruct((NP, D), jnp.float32),
          jax.ShapeDtypeStruct((N, 1), jnp.float32),
          jax.ShapeDtypeStruct((N, 1), jnp.float32),
      ),
  )(node2, Wfc1, bfc1.reshape(1, -1),
    jnp.concatenate([Wfc2, jnp.zeros((6, Wfc2.shape[1]), jnp.float32)], axis=0),
    jnp.concatenate([bfc2, jnp.zeros((6,), jnp.float32)]).reshape(1, 8), u)


def _tc2b_body(node2_ref, lp_ref, ln_ref, graphf_ref, nfm_ref, nfs_ref,
               noise_ref, wc1_ref, bc1_ref, wc2_ref, bc2_ref, label_ref,
               noisyg_ref, kl_ref, cls_ref):
  node2 = node2_ref[...]
  lp = lp_ref[...]
  ln = ln_ref[...]
  nf_mean = nfm_ref[...]
  nf_std = nfs_ref[...]

  noisy_mean = lp * node2 + ln * nf_mean
  noisy_std = ln * nf_std
  noisy_node = noisy_mean + noise_ref[...] * noisy_std
  noisyg = jnp.sum(noisy_node, axis=0, keepdims=True)
  noisyg_ref[...] = noisyg

  denom = (nf_std + EPS) ** 2
  t1 = 0.5 * (noisy_std * noisy_std) / denom
  dmean = noisy_mean - nf_mean
  t2 = jnp.sum(dmean * dmean / denom, axis=0, keepdims=True)
  kl = jnp.sum(t1) / (N * D) + jnp.sum(t2) / D
  kl_ref[...] = kl * jnp.ones((1, 1), jnp.float32)

  v2 = jnp.concatenate([graphf_ref[...], noisyg], axis=0)
  h = jnp.maximum(_dotT(v2, wc1_ref[...]) + bc1_ref[...], 0.0)
  o = jnp.sum(h * wc2_ref[...], axis=1, keepdims=True) + bc2_ref[...]
  o = jnp.maximum(o, 0.0)
  cd = o - label_ref[...]
  cls_ref[...] = jnp.sum(cd * cd, axis=0, keepdims=True)


def _tc2b(node2, lp, ln, graphf, nfm, nfs, noise, Wc1, bc1, Wc2, bc2, label):
  return pl.pallas_call(
      _tc2b_body,
      out_shape=(
          jax.ShapeDtypeStruct((1, D), jnp.float32),
          jax.ShapeDtypeStruct((1, 1), jnp.float32),
          jax.ShapeDtypeStruct((1, 1), jnp.float32),
      ),
  )(node2, lp, ln, graphf, nfm, nfs, noise,
    Wc1, bc1.reshape(1, -1), Wc2, bc2.reshape(1, 1),
    jnp.broadcast_to(label.reshape(1, 1), (2, 1)))


def _tc3_body(adj_ref, out_ref):
  sums = jnp.sum(adj_ref[...], axis=0, keepdims=True)
  t = sums[0:1, 0:1]
  sg = sums[0:1, 1:2]
  sd = sums[0:1, 2:3]
  n00 = t
  n01 = sg - t
  n10 = sd - t
  n11 = float(E) - sg - sd + t
  d0 = n00 / jnp.maximum(jnp.abs(n00) + jnp.abs(n01), 1e-12) - 1.0
  d1 = n11 / jnp.maximum(jnp.abs(n10) + jnp.abs(n11), 1e-12) - 1.0
  out_ref[...] = (d0 * d0 + d1 * d1) * 0.5


def _tc3(adj):
  return pl.pallas_call(
      _tc3_body,
      out_shape=jax.ShapeDtypeStruct((1, 1), jnp.float32),
  )(adj)


@jax.jit
def kernel(features, edges, label, W1l, b1l, W1r, W2l, b2l, W2r,
           Wfc1, bfc1, Wfc2, bfc2, Wc1, bc1, Wc2, bc2):
  src = edges[0]
  dst = edges[1]
  u = jax.random.uniform(jax.random.key(42), (N, 2), minval=1e-10, maxval=1.0)
  noise = jax.random.uniform(jax.random.key(43), (N, D))

  (cnt128,) = _make_count()(dst)
  cntp = cnt128[:, :, 0:16]
  (agg1,) = _make_segsum(D)(features, src, dst)
  node1 = _tc1(features, agg1, cntp, W1l, b1l, W1r)
  (agg2,) = _make_segsum(D)(node1, src, dst)
  node2, graphf, nfm, nfs = _tc2a(node1, agg2, cntp, W2l, b2l, W2r)
  asg16, lp, ln = _tcasg(node2, Wfc1, bfc1, Wfc2, bfc2, u)
  noisyg, kl, cls = _tc2b(node2, lp, ln, graphf, nfm, nfs, noise,
                          Wc1, bc1, Wc2, bc2, label)
  (adj,) = _make_adj()(asg16, src, dst)
  pp = _tc3(adj.reshape(NW, 16))

  return (graphf, noisyg, noisyg, kl[0, 0], cls[0, 0], pp[0, 0])
